# baseline (speedup 1.0000x reference)
import functools

import numpy as np
import jax
import jax.numpy as jnp
from jax import lax
from jax.experimental import pallas as pl
from jax.experimental.pallas import tpu as pltpu
from jax.experimental.pallas import tpu_sc as plsc

B, C, T = 4, 768, 2048
K = 3
FP = 1152
T2 = 4160
R = 128
NROWS = B * C
TT = 512


def _dft_consts():
    a = np.arange(FP, dtype=np.float64)[:, None]
    b = np.arange(FP, dtype=np.float64)[None, :]
    ang = 2.0 * np.pi * ((a * b) % T) / T
    valid = (a <= T // 2) & (b <= T // 2)
    cosm = np.where(valid, np.cos(ang), 0.0)
    sinm = np.where(valid, np.sin(ang), 0.0)
    wf = np.where((a == 0) | (a == T // 2), 1.0, 2.0)
    icos = np.where(valid, wf * np.cos(ang) / (T * T), 0.0)
    isin = np.where(valid, wf * np.sin(ang) / (T * T), 0.0)
    return (cosm.astype(np.float32), sinm.astype(np.float32),
            icos.astype(np.float32), isin.astype(np.float32))


_COSM, _SINM, _ICOS, _ISIN = _dft_consts()


def _proj_body(wq, bq, wk, bk, wv, bv, x_q, x_k, x_v, q_out, k_out, v_out):
    q_out[0] = jnp.dot(wq[...], x_q[0], preferred_element_type=jnp.float32) + bq[...]
    k_out[0] = jnp.dot(wk[...], x_k[0], preferred_element_type=jnp.float32) + bk[...]
    v_out[0] = jnp.dot(wv[...], x_v[0], preferred_element_type=jnp.float32) + bv[...]


def _proj(query, key_in, value, Wq, bq, Wk, bk, Wv, bv):
    full = pl.BlockSpec((C, C), lambda b, t: (0, 0))
    bias = pl.BlockSpec((C, 1), lambda b, t: (0, 0))
    xblk = pl.BlockSpec((1, C, TT), lambda b, t: (b, 0, t))
    return pl.pallas_call(
        _proj_body,
        grid=(B, T // TT),
        in_specs=[full, bias, full, bias, full, bias, xblk, xblk, xblk],
        out_specs=[xblk, xblk, xblk],
        out_shape=[jax.ShapeDtypeStruct((B, C, T), jnp.float32)] * 3,
        compiler_params=pltpu.CompilerParams(
            dimension_semantics=("arbitrary", "arbitrary")),
    )(Wq, bq.reshape(C, 1), Wk, bk.reshape(C, 1), Wv, bv.reshape(C, 1),
      query, key_in, value)


def _dup_body(v_in, v2_out):
    v2_out[0, :, 0:T] = v_in[0]
    v2_out[0, :, T:2 * T] = v_in[0]
    v2_out[0, :, 2 * T:T2] = v_in[0, :, 0:T2 - 2 * T]


def _dup(v):
    return pl.pallas_call(
        _dup_body,
        grid=(B,),
        in_specs=[pl.BlockSpec((1, C, T), lambda b: (b, 0, 0))],
        out_specs=pl.BlockSpec((1, C, T2), lambda b: (b, 0, 0)),
        out_shape=jax.ShapeDtypeStruct((B, C, T2), jnp.float32),
    )(v)


def _acsel_body(qe_ref, qo_ref, ke_ref, ko_ref, cosm, sinm, icos, isin,
                offs_out, wsel_out):
    hi = jax.lax.Precision.HIGHEST
    H = T // 2
    qe, qo, ke, ko = qe_ref[...], qo_ref[...], ke_ref[...], ko_ref[...]
    fqr = jnp.dot(qe, cosm[...], preferred_element_type=jnp.float32, precision=hi)
    fqs = jnp.dot(qo, sinm[...], preferred_element_type=jnp.float32, precision=hi)
    fkr = jnp.dot(ke, cosm[...], preferred_element_type=jnp.float32, precision=hi)
    fks = jnp.dot(ko, sinm[...], preferred_element_type=jnp.float32, precision=hi)
    pr = fqr * fkr + fqs * fks
    pi = fqr * fks - fqs * fkr
    acc = jnp.dot(pr, icos[...], preferred_element_type=jnp.float32, precision=hi)
    acs = jnp.dot(pi, isin[...], preferred_element_type=jnp.float32, precision=hi)
    col = lax.broadcasted_iota(jnp.int32, (R, FP), 1)
    neg = jnp.float32(-jnp.inf)
    mL = jnp.where(col <= H, acc - acs, neg)
    mR = jnp.where((col >= 1) & (col <= H - 1), acc + acs, neg)
    vals, lags = [], []
    for _ in range(K):
        vL = jnp.max(mL, axis=-1, keepdims=True)
        lL = jnp.min(jnp.where(mL == vL, col, T), axis=-1, keepdims=True)
        vR = jnp.max(mR, axis=-1, keepdims=True)
        cR = jnp.max(jnp.where(mR == vR, col, -1), axis=-1, keepdims=True)
        lR = T - cR
        pickL = (vL > vR) | ((vL == vR) & (lL < lR))
        v1 = jnp.where(pickL, vL, vR)
        l1 = jnp.where(pickL, lL, lR)
        vals.append(v1)
        lags.append(l1)
        mL = jnp.where(pickL & (col == lL), neg, mL)
        mR = jnp.where((~pickL) & (col == cR), neg, mR)
    m = vals[0]
    z = (jnp.sum(jnp.exp(mL - m), axis=-1, keepdims=True)
         + jnp.sum(jnp.exp(mR - m), axis=-1, keepdims=True))
    zsel = sum(jnp.exp(v - m) for v in vals)
    z = z + zsel
    lag = jnp.concatenate(lags, axis=1)
    val = jnp.concatenate(vals, axis=1)
    offs_out[...] = T - lag
    wsel_out[...] = jnp.exp(val - m) / z


def _fold(x2d):
    col = jnp.arange(T)
    xr = jnp.where((col == 0) | (col == T // 2), 0.0,
                   jnp.roll(x2d[:, ::-1], 1, axis=1))
    return (x2d + xr)[:, :FP], (x2d - xr)[:, :FP]


def _acsel(q2d, k2d):
    rows = pl.BlockSpec((R, FP), lambda r: (r, 0))
    mat = pl.BlockSpec((FP, FP), lambda r: (0, 0))
    sel = pl.BlockSpec((R, K), lambda r: (r, 0))
    qe, qo = _fold(q2d)
    ke, ko = _fold(k2d)
    return pl.pallas_call(
        _acsel_body,
        grid=(NROWS // R,),
        in_specs=[rows, rows, rows, rows, mat, mat, mat, mat],
        out_specs=[sel, sel],
        out_shape=[jax.ShapeDtypeStruct((NROWS, K), jnp.int32),
                   jax.ShapeDtypeStruct((NROWS, K), jnp.float32)],
        compiler_params=pltpu.CompilerParams(
            vmem_limit_bytes=100 * 1024 * 1024),
    )(qe, qo, ke, ko, jnp.asarray(_COSM), jnp.asarray(_SINM),
      jnp.asarray(_ICOS), jnp.asarray(_ISIN))


_NC, _NS = 2, 16
_NW = _NC * _NS
_RPW = NROWS // _NW


def _roll_body(v2_hbm, offs_hbm, out_hbm, offs_v, row_v, obuf_v, gsem, ssem):
    wid = lax.axis_index("s") * _NC + lax.axis_index("c")
    base = wid * _RPW
    pltpu.sync_copy(offs_hbm.at[:, pl.ds(base, _RPW)], offs_v)
    pltpu.async_copy(v2_hbm.at[base], row_v.at[0], gsem)

    def row_body(j, _):
        rid = base + j
        cur = j & 1
        @pl.when(j + 1 < _RPW)
        def _():
            pltpu.async_copy(v2_hbm.at[rid + 1], row_v.at[1 - cur], gsem)
        pltpu.make_async_copy(v2_hbm.at[rid], row_v.at[cur], gsem).wait()
        @pl.when(j >= 2)
        def _():
            pltpu.make_async_copy(out_hbm.at[:, 0], obuf_v.at[cur], ssem).wait()
        for i in range(K):
            chunk = offs_v[i, pl.ds((j >> 4) << 4, 16)]
            lane = j & 15
            sel = jnp.where(lax.iota(jnp.int32, 16) == lane, chunk, 0)
            off = lax.reduce_max(sel, (0,))

            def cp(j2, _):
                obuf_v[cur, i, pl.ds(j2 * 16, 16)] = row_v[cur, pl.ds(off + j2 * 16, 16)]
                return 0

            lax.fori_loop(0, T // 16, cp, 0, unroll=8)
            pltpu.async_copy(obuf_v.at[cur, i], out_hbm.at[i, rid], ssem)
        return 0

    lax.fori_loop(0, _RPW, row_body, 0)
    pltpu.make_async_copy(out_hbm.at[:, 0], obuf_v.at[0], ssem).wait()
    pltpu.make_async_copy(out_hbm.at[:, 0], obuf_v.at[1], ssem).wait()


@functools.partial(jax.jit, static_argnums=())
def _roll_sc(v2_rows, offs_t):
    mesh = plsc.VectorSubcoreMesh(core_axis_name="c", subcore_axis_name="s")
    return pl.kernel(
        _roll_body,
        out_type=jax.ShapeDtypeStruct((K, NROWS, T), jnp.float32),
        mesh=mesh,
        compiler_params=pltpu.CompilerParams(use_tc_tiling_on_sc=False,
                                             needs_layout_passes=False),
        scratch_types=[
            pltpu.VMEM((K, _RPW), jnp.int32),
            pltpu.VMEM((2, T2), jnp.float32),
            pltpu.VMEM((2, K, T), jnp.float32),
            pltpu.SemaphoreType.DMA,
            pltpu.SemaphoreType.DMA,
        ],
    )(v2_rows, offs_t)


def _final_body(wf, bf, wsel, rolled, out):
    ws = wsel[0]
    acc = bf[...]
    for i in range(K):
        wfi = wf[:, i * C:(i + 1) * C] * ws[:, i][None, :]
        acc = acc + jnp.dot(wfi, rolled[i, 0],
                            preferred_element_type=jnp.float32)
    out[0] = acc


def _final(Wf, bf, wsel, rolled):
    return pl.pallas_call(
        _final_body,
        grid=(B, T // TT),
        in_specs=[
            pl.BlockSpec((C, K * C), lambda b, t: (0, 0)),
            pl.BlockSpec((C, 1), lambda b, t: (0, 0)),
            pl.BlockSpec((1, C, K), lambda b, t: (b, 0, 0)),
            pl.BlockSpec((K, 1, C, TT), lambda b, t: (0, b, 0, t)),
        ],
        out_specs=pl.BlockSpec((1, C, TT), lambda b, t: (b, 0, t)),
        out_shape=jax.ShapeDtypeStruct((B, C, T), jnp.float32),
        compiler_params=pltpu.CompilerParams(
            dimension_semantics=("arbitrary", "arbitrary")),
    )(Wf, bf.reshape(C, 1), wsel, rolled)


def kernel(query, key_in, value, Wq, bq, Wk, bk, Wv, bv, Wf, bf):
    q, k, v = _proj(query, key_in, value, Wq, bq, Wk, bk, Wv, bv)
    v2 = _dup(v)
    offs, wsel = _acsel(q.reshape(NROWS, T), k.reshape(NROWS, T))
    rolled = _roll_sc(v2.reshape(NROWS, T2), offs.T)
    return _final(Wf, bf, wsel.reshape(B, C, K),
                  rolled.reshape(K, B, C, T))
